# baseline (speedup 1.0000x reference)
import jax
import jax.numpy as jnp
from jax import lax
from jax.experimental import pallas as pl
from jax.experimental.pallas import tpu as pltpu
from jax.experimental.pallas import tpu_sc as plsc

B, Q, D, E, NBINS = 64, 16, 8192, 300, 30

SC_ROWS = 2048
D0 = D - SC_ROWS
NSTREAM = 4
DC = D0 // NSTREAM
B_PER_W = 2
SC_CHUNK = 256
_NFULL = E // 16
_TAIL = E - 16
_NPAD = 34


def _tc_hist_step(q_ref, *rest):
    d_refs = rest[:NSTREAM]
    out_ref = rest[NSTREAM]
    eps = 1e-8
    q = q_ref[0]

    qss = jnp.sum(q * q, axis=1, keepdims=True)
    qn = q * (1.0 / jnp.maximum(jnp.sqrt(qss), eps))

    counts = [jnp.zeros((Q, 1), jnp.float32)] * NBINS
    for s in range(NSTREAM):
        d = d_refs[s][0]
        dss = jnp.sum(d * d, axis=1, keepdims=True)
        inv_dn = 1.0 / jnp.maximum(jnp.sqrt(dss), eps)
        inter = jax.lax.dot_general(
            qn, d, (((1,), (1,)), ((), ())),
            preferred_element_type=jnp.float32,
            precision=jax.lax.Precision.DEFAULT)
        inter = inter * inv_dn.reshape(1, DC)
        y = jnp.floor((inter + 1.0) * (0.5 * NBINS))
        y = jnp.clip(y, 0.0, NBINS - 1.0)
        for k in range(NBINS):
            counts[k] = counts[k] + jnp.sum(
                jnp.where(y == float(k), 1.0, 0.0), axis=1, keepdims=True)
    out_ref[0] = jnp.concatenate(counts, axis=1)


def _newton_rsqrt(x):
    m = x
    s = jnp.ones_like(x)
    for p in (32, 16, 8, 4, 2, 1):
        big = m >= (2.0 ** p)
        m = jnp.where(big, m * (2.0 ** -p), m)
        s = jnp.where(big, s * (2.0 ** (-p / 2)), s)
        small = m < (2.0 ** -p)
        m = jnp.where(small, m * (2.0 ** p), m)
        s = jnp.where(small, s * (2.0 ** (p / 2)), s)
    y = jnp.full_like(x, 1.0)
    for _ in range(6):
        y = y * (1.5 - 0.5 * m * y * y)
    return y * s


def _sc_hist(q_hbm, doc_hbm, out_hbm, qbuf, dbuf, hist, pad):
    cax = lax.axis_index("c")
    sax = lax.axis_index("s")
    wid = sax * 2 + cax
    lanes = lax.iota(jnp.int32, 16)
    zeros16 = jnp.zeros((16,), jnp.float32)
    tailmask = lanes >= (16 - (E - 16 * _NFULL))

    for slot in range(_NPAD):
        pad[slot, pl.ds(16, 16)] = zeros16

    def hsum(slot, v):
        y = v
        for sh in (8, 4, 2, 1):
            pad[slot, pl.ds(0, 16)] = y
            y = y + pad[slot, pl.ds(sh, 16)]
        return y[0]

    def row_slices():
        for cc in range(_NFULL):
            yield False, pl.ds(cc * 16, 16)
        yield True, pl.ds(_TAIL, 16)

    for bb in range(B_PER_W):
        b = wid * B_PER_W + bb
        pltpu.sync_copy(q_hbm.at[b], qbuf)

        ssqv = zeros16
        for qi in range(Q):
            acc = zeros16
            for is_tail, sl in row_slices():
                qc = qbuf[qi, sl]
                sq = qc * qc
                acc = acc + (jnp.where(tailmask, sq, 0.0) if is_tail else sq)
            ssqv = jnp.where(lanes == qi, hsum(qi % _NPAD, acc), ssqv)
        invq = jnp.minimum(_newton_rsqrt(jnp.maximum(ssqv, 1e-18)), 1e8)

        def chunk_body(ch, counts):
            pltpu.sync_copy(
                doc_hbm.at[b, pl.ds(D0 + ch * SC_CHUNK, SC_CHUNK)], dbuf)

            def pair_body(p, counts):
                r0 = p * 2
                acc0 = [zeros16] * Q
                acc1 = [zeros16] * Q
                ssd0 = zeros16
                ssd1 = zeros16
                for is_tail, sl in row_slices():
                    d0 = dbuf[r0, sl]
                    d1 = dbuf[r0 + 1, sl]
                    if is_tail:
                        ssd0 = ssd0 + jnp.where(tailmask, d0 * d0, 0.0)
                        ssd1 = ssd1 + jnp.where(tailmask, d1 * d1, 0.0)
                    else:
                        ssd0 = ssd0 + d0 * d0
                        ssd1 = ssd1 + d1 * d1
                    for qi in range(Q):
                        qc = qbuf[qi, sl]
                        if is_tail:
                            acc0[qi] = acc0[qi] + jnp.where(
                                tailmask, qc * d0, 0.0)
                            acc1[qi] = acc1[qi] + jnp.where(
                                tailmask, qc * d1, 0.0)
                        else:
                            acc0[qi] = acc0[qi] + qc * d0
                            acc1[qi] = acc1[qi] + qc * d1

                for ri, (accs, ssd) in enumerate(((acc0, ssd0),
                                                  (acc1, ssd1))):
                    dotv = zeros16
                    for qi in range(Q):
                        dotv = jnp.where(lanes == qi,
                                         hsum(ri * 17 + qi, accs[qi]), dotv)
                    ssr = jnp.full((16,), hsum(ri * 17 + 16, ssd),
                                   jnp.float32)
                    invn = jnp.minimum(
                        _newton_rsqrt(jnp.maximum(ssr, 1e-18)), 1e8)
                    inter = dotv * invq * invn
                    v = (inter + 1.0) * (0.5 * NBINS)
                    v = jnp.minimum(jnp.maximum(v, 0.0), float(NBINS - 1))
                    yi = v.astype(jnp.int32)
                    counts = tuple(
                        counts[k] + jnp.where(yi == k, 1.0, 0.0)
                        for k in range(NBINS))
                return counts
            return lax.fori_loop(0, SC_CHUNK // 2, pair_body, counts)

        counts = lax.fori_loop(0, SC_ROWS // SC_CHUNK, chunk_body,
                               (zeros16,) * NBINS)
        for k in range(NBINS):
            hist[pl.ds(k * 16, 16)] = counts[k]
        pltpu.sync_copy(hist, out_hbm.at[b])


def _combine_step(ht_ref, hs_ref, q_ref, w1_ref, b1_ref, w2_ref, b2_ref,
                  w3_ref, b3_ref, wg_ref, bg_ref, out_ref):
    h = jnp.log1p(ht_ref[0] + hs_ref[0])
    w1 = w1_ref[...]
    zcols = []
    for j in range(5):
        zcols.append(jnp.sum(h * w1[j:j + 1, :], axis=1, keepdims=True))
    z = jnp.tanh(jnp.concatenate(zcols, axis=1) + b1_ref[...])
    z = jnp.tanh(jnp.sum(z * w2_ref[...], axis=1, keepdims=True)
                 + b2_ref[...])
    z = jnp.tanh(z * w3_ref[...] + b3_ref[...])

    q = q_ref[0]
    gate = jnp.sum(q * wg_ref[...], axis=1, keepdims=True) + bg_ref[...]
    gate = gate - jnp.max(gate, axis=0, keepdims=True)
    gate = jnp.exp(gate)
    gate = gate / jnp.sum(gate, axis=0, keepdims=True)

    out_ref[...] = jnp.sum(z * gate).reshape(1, 1, 1)


@jax.jit
def kernel(query, document, W1, b1, W2, b2, W3, b3, Wg, bg):
    b1r = b1.reshape(1, 5)
    b2r = b2.reshape(1, 1)
    b3r = b3.reshape(1, 1)
    bgr = bg.reshape(1, 1)

    def dspec(s):
        return pl.BlockSpec((1, DC, E), lambda b, s=s: (b, s, 0))

    hist_sc = pl.kernel(
        _sc_hist,
        out_type=jax.ShapeDtypeStruct((B, Q * NBINS), jnp.float32),
        mesh=plsc.VectorSubcoreMesh(core_axis_name="c", subcore_axis_name="s"),
        scratch_types=[
            pltpu.VMEM((Q, E), jnp.float32),
            pltpu.VMEM((SC_CHUNK, E), jnp.float32),
            pltpu.VMEM((Q * NBINS,), jnp.float32),
            pltpu.VMEM((_NPAD, 32), jnp.float32),
        ],
    )(query, document)

    hist_tc = pl.pallas_call(
        _tc_hist_step,
        grid=(B,),
        in_specs=[
            pl.BlockSpec((1, Q, E), lambda b: (b, 0, 0)),
            *[dspec(s) for s in range(NSTREAM)],
        ],
        out_specs=pl.BlockSpec((1, Q, NBINS), lambda b: (b, 0, 0)),
        out_shape=jax.ShapeDtypeStruct((B, Q, NBINS), jnp.float32),
    )(query, *([document] * NSTREAM))

    hist_sc = hist_sc.reshape(B, NBINS, Q).transpose(0, 2, 1)

    out = pl.pallas_call(
        _combine_step,
        grid=(B,),
        in_specs=[
            pl.BlockSpec((1, Q, NBINS), lambda b: (b, 0, 0)),
            pl.BlockSpec((1, Q, NBINS), lambda b: (b, 0, 0)),
            pl.BlockSpec((1, Q, E), lambda b: (b, 0, 0)),
            pl.BlockSpec((5, NBINS), lambda b: (0, 0)),
            pl.BlockSpec((1, 5), lambda b: (0, 0)),
            pl.BlockSpec((1, 5), lambda b: (0, 0)),
            pl.BlockSpec((1, 1), lambda b: (0, 0)),
            pl.BlockSpec((1, 1), lambda b: (0, 0)),
            pl.BlockSpec((1, 1), lambda b: (0, 0)),
            pl.BlockSpec((1, E), lambda b: (0, 0)),
            pl.BlockSpec((1, 1), lambda b: (0, 0)),
        ],
        out_specs=pl.BlockSpec((1, 1, 1), lambda b: (b, 0, 0)),
        out_shape=jax.ShapeDtypeStruct((B, 1, 1), jnp.float32),
    )(hist_tc, hist_sc, query, W1, b1r, W2, b2r, W3, b3r, Wg, bgr)
    return out[:, 0, 0]


# speedup vs baseline: 1.8327x; 1.8327x over previous
import jax
import jax.numpy as jnp
from jax.experimental import pallas as pl

B, Q, D, E, NBINS = 64, 16, 8192, 300, 30
NSTREAM = 4
DC = D // NSTREAM


def _drmm_step(q_ref, *rest):
    d_refs = rest[:NSTREAM]
    (w1_ref, b1_ref, w2_ref, b2_ref, w3_ref, b3_ref, wg_ref, bg_ref,
     out_ref) = rest[NSTREAM:]
    eps = 1e-8
    q = q_ref[0]

    qss = jnp.sum(q * q, axis=1, keepdims=True)
    qn = q * (1.0 / jnp.maximum(jnp.sqrt(qss), eps))

    counts = [jnp.zeros((Q, 1), jnp.float32)] * NBINS
    for s in range(NSTREAM):
        d = d_refs[s][0]
        dss = jnp.sum(d * d, axis=1, keepdims=True)
        inv_dn = 1.0 / jnp.maximum(jnp.sqrt(dss), eps)
        inter = jax.lax.dot_general(
            qn, d, (((1,), (1,)), ((), ())),
            preferred_element_type=jnp.float32,
            precision=jax.lax.Precision.DEFAULT)
        inter = inter * inv_dn.reshape(1, DC)
        y = jnp.floor((inter + 1.0) * (0.5 * NBINS))
        y = jnp.clip(y, 0.0, NBINS - 1.0)
        for k in range(NBINS):
            counts[k] = counts[k] + jnp.sum(
                jnp.where(y == float(k), 1.0, 0.0), axis=1, keepdims=True)
    h = jnp.concatenate(counts, axis=1)

    h = jnp.log1p(h)
    w1 = w1_ref[...]
    zcols = []
    for j in range(5):
        zcols.append(jnp.sum(h * w1[j:j + 1, :], axis=1, keepdims=True))
    z = jnp.tanh(jnp.concatenate(zcols, axis=1) + b1_ref[...])
    z = jnp.tanh(jnp.sum(z * w2_ref[...], axis=1, keepdims=True)
                 + b2_ref[...])
    z = jnp.tanh(z * w3_ref[...] + b3_ref[...])

    gate = jnp.sum(q * wg_ref[...], axis=1, keepdims=True) + bg_ref[...]
    gate = gate - jnp.max(gate, axis=0, keepdims=True)
    gate = jnp.exp(gate)
    gate = gate / jnp.sum(gate, axis=0, keepdims=True)

    out_ref[...] = jnp.sum(z * gate).reshape(1, 1, 1)


@jax.jit
def kernel(query, document, W1, b1, W2, b2, W3, b3, Wg, bg):
    b1r = b1.reshape(1, 5)
    b2r = b2.reshape(1, 1)
    b3r = b3.reshape(1, 1)
    bgr = bg.reshape(1, 1)

    def dspec(s):
        return pl.BlockSpec((1, DC, E), lambda b, s=s: (b, s, 0))

    grid = (B,)
    out = pl.pallas_call(
        _drmm_step,
        grid=grid,
        in_specs=[
            pl.BlockSpec((1, Q, E), lambda b: (b, 0, 0)),
            *[dspec(s) for s in range(NSTREAM)],
            pl.BlockSpec((5, NBINS), lambda b: (0, 0)),
            pl.BlockSpec((1, 5), lambda b: (0, 0)),
            pl.BlockSpec((1, 5), lambda b: (0, 0)),
            pl.BlockSpec((1, 1), lambda b: (0, 0)),
            pl.BlockSpec((1, 1), lambda b: (0, 0)),
            pl.BlockSpec((1, 1), lambda b: (0, 0)),
            pl.BlockSpec((1, E), lambda b: (0, 0)),
            pl.BlockSpec((1, 1), lambda b: (0, 0)),
        ],
        out_specs=pl.BlockSpec((1, 1, 1), lambda b: (b, 0, 0)),
        out_shape=jax.ShapeDtypeStruct((B, 1, 1), jnp.float32),
    )(query, *([document] * NSTREAM), W1, b1r, W2, b2r, W3, b3r, Wg, bgr)
    return out[:, 0, 0]


# speedup vs baseline: 1.8610x; 1.0155x over previous
import jax
import jax.numpy as jnp
from jax.experimental import pallas as pl

B, Q, D, E, NBINS = 64, 16, 8192, 300, 30
BPS = 2


def _drmm_step(q_ref, d_ref, w1_ref, b1_ref, w2_ref, b2_ref, w3_ref, b3_ref,
               wg_ref, bg_ref, out_ref):
    eps = 1e-8
    for bi in range(BPS):
        q = q_ref[bi]
        qss = jnp.sum(q * q, axis=1, keepdims=True)
        qn = q * (1.0 / jnp.maximum(jnp.sqrt(qss), eps))

        counts = [jnp.zeros((Q, 1), jnp.float32)] * NBINS
        for s in range(4):
            d = d_ref[bi, pl.ds(s * (D // 4), D // 4)]
            dss = jnp.sum(d * d, axis=1, keepdims=True)
            inv_dn = 1.0 / jnp.maximum(jnp.sqrt(dss), eps)
            inter = jax.lax.dot_general(
                qn, d, (((1,), (1,)), ((), ())),
                preferred_element_type=jnp.float32,
                precision=jax.lax.Precision.DEFAULT)
            inter = inter * inv_dn.reshape(1, D // 4)
            y = jnp.floor((inter + 1.0) * (0.5 * NBINS))
            y = jnp.clip(y, 0.0, NBINS - 1.0)
            for k in range(NBINS):
                counts[k] = counts[k] + jnp.sum(
                    jnp.where(y == float(k), 1.0, 0.0), axis=1,
                    keepdims=True)
        h = jnp.concatenate(counts, axis=1)

        h = jnp.log1p(h)
        w1 = w1_ref[...]
        zcols = []
        for j in range(5):
            zcols.append(jnp.sum(h * w1[j:j + 1, :], axis=1, keepdims=True))
        z = jnp.tanh(jnp.concatenate(zcols, axis=1) + b1_ref[...])
        z = jnp.tanh(jnp.sum(z * w2_ref[...], axis=1, keepdims=True)
                     + b2_ref[...])
        z = jnp.tanh(z * w3_ref[...] + b3_ref[...])

        gate = jnp.sum(q * wg_ref[...], axis=1, keepdims=True) + bg_ref[...]
        gate = gate - jnp.max(gate, axis=0, keepdims=True)
        gate = jnp.exp(gate)
        gate = gate / jnp.sum(gate, axis=0, keepdims=True)

        out_ref[bi] = jnp.sum(z * gate).reshape(1, 1)


@jax.jit
def kernel(query, document, W1, b1, W2, b2, W3, b3, Wg, bg):
    b1r = b1.reshape(1, 5)
    b2r = b2.reshape(1, 1)
    b3r = b3.reshape(1, 1)
    bgr = bg.reshape(1, 1)

    out = pl.pallas_call(
        _drmm_step,
        grid=(B // BPS,),
        in_specs=[
            pl.BlockSpec((BPS, Q, E), lambda b: (b, 0, 0)),
            pl.BlockSpec((BPS, D, E), lambda b: (b, 0, 0)),
            pl.BlockSpec((5, NBINS), lambda b: (0, 0)),
            pl.BlockSpec((1, 5), lambda b: (0, 0)),
            pl.BlockSpec((1, 5), lambda b: (0, 0)),
            pl.BlockSpec((1, 1), lambda b: (0, 0)),
            pl.BlockSpec((1, 1), lambda b: (0, 0)),
            pl.BlockSpec((1, 1), lambda b: (0, 0)),
            pl.BlockSpec((1, E), lambda b: (0, 0)),
            pl.BlockSpec((1, 1), lambda b: (0, 0)),
        ],
        out_specs=pl.BlockSpec((BPS, 1, 1), lambda b: (b, 0, 0)),
        out_shape=jax.ShapeDtypeStruct((B, 1, 1), jnp.float32),
    )(query, document, W1, b1r, W2, b2r, W3, b3r, Wg, bgr)
    return out[:, 0, 0]
